# baseline (speedup 1.0000x reference)
import functools

import jax
import jax.numpy as jnp
from jax import lax
from jax.experimental import pallas as pl
from jax.experimental.pallas import tpu as pltpu
from jax.experimental.pallas import tpu_sc as plsc


_NBUF = 12
_CHUNK = 8


def _make_gather(vocab: int, d_model: int, batch: int, seq: int):
    info = plsc.get_sparse_core_info()
    nc, ns = info.num_cores, info.num_subcores
    nw = nc * ns
    n_idx = batch * seq
    per_w = n_idx // nw
    chunk = _CHUNK
    n_chunks = per_w // chunk
    w_per_row = seq // per_w

    mesh = plsc.VectorSubcoreMesh(core_axis_name="c", subcore_axis_name="s")

    @functools.partial(
        pl.kernel,
        mesh=mesh,
        out_type=jax.ShapeDtypeStruct((n_idx, d_model), jnp.float32),
        scratch_types=(
            [pltpu.VMEM((per_w,), jnp.int32)]
            + [pltpu.VMEM((chunk, d_model), jnp.float32)
               for _ in range(_NBUF)]
            + [pltpu.SemaphoreType.DMA for _ in range(2 * _NBUF)]
        ),
    )
    def gather_kernel(idx_hbm, table_hbm, out_hbm, idx_v, *scratch):
        bufs = scratch[:_NBUF]
        sem_g = scratch[_NBUF:2 * _NBUF]
        sem_s = scratch[2 * _NBUF:]
        wid = lax.axis_index("s") * nc + lax.axis_index("c")
        base = wid * per_w
        row = wid // w_per_row
        col = (wid % w_per_row) * per_w
        pltpu.sync_copy(idx_hbm.at[row, pl.ds(col, per_w)], idx_v)

        def start_g(i):
            b = i % _NBUF
            return pltpu.async_copy(
                table_hbm.at[idx_v.at[pl.ds(i * chunk, chunk)]], bufs[b],
                sem_g[b])

        def start_s(i):
            b = i % _NBUF
            return pltpu.async_copy(
                bufs[b], out_hbm.at[pl.ds(base + i * chunk, chunk)],
                sem_s[b])

        gathers = [None] * n_chunks
        scatters = [None] * n_chunks
        for j in range(min(_NBUF, n_chunks)):
            gathers[j] = start_g(j)
        for i in range(n_chunks):
            gathers[i].wait()
            scatters[i] = start_s(i)
            nxt = i - 1 + _NBUF
            if i >= 1 and nxt < n_chunks:
                scatters[i - 1].wait()
                gathers[nxt] = start_g(nxt)
        for i in range(max(0, n_chunks - _NBUF), n_chunks):
            scatters[i].wait()

    return gather_kernel


def kernel(input_ids, attention_mask, position_ids, labels, embed_table):
    vocab, d_model = embed_table.shape
    b, s = input_ids.shape
    gather = _make_gather(vocab, d_model, b, s)
    rows = gather(input_ids, embed_table)
    return (rows.reshape(b, s, d_model), attention_mask, position_ids, labels)


# speedup vs baseline: 1.0131x; 1.0131x over previous
import functools

import jax
import jax.numpy as jnp
from jax import lax
from jax.experimental import pallas as pl
from jax.experimental.pallas import tpu as pltpu
from jax.experimental.pallas import tpu_sc as plsc


_NBUF = 7
_CHUNK = 16


def _make_gather(vocab: int, d_model: int, batch: int, seq: int):
    info = plsc.get_sparse_core_info()
    nc, ns = info.num_cores, info.num_subcores
    nw = nc * ns
    n_idx = batch * seq
    per_w = n_idx // nw
    chunk = _CHUNK
    n_chunks = per_w // chunk
    w_per_row = seq // per_w

    mesh = plsc.VectorSubcoreMesh(core_axis_name="c", subcore_axis_name="s")

    @functools.partial(
        pl.kernel,
        mesh=mesh,
        out_type=jax.ShapeDtypeStruct((n_idx, d_model), jnp.float32),
        scratch_types=(
            [pltpu.VMEM((per_w,), jnp.int32)]
            + [pltpu.VMEM((chunk, d_model), jnp.float32)
               for _ in range(_NBUF)]
            + [pltpu.SemaphoreType.DMA for _ in range(2 * _NBUF)]
        ),
    )
    def gather_kernel(idx_hbm, table_hbm, out_hbm, idx_v, *scratch):
        bufs = scratch[:_NBUF]
        sem_g = scratch[_NBUF:2 * _NBUF]
        sem_s = scratch[2 * _NBUF:]
        wid = lax.axis_index("s") * nc + lax.axis_index("c")
        base = wid * per_w
        row = wid // w_per_row
        col = (wid % w_per_row) * per_w
        pltpu.sync_copy(idx_hbm.at[row, pl.ds(col, per_w)], idx_v)

        def start_g(i):
            b = i % _NBUF
            return pltpu.async_copy(
                table_hbm.at[idx_v.at[pl.ds(i * chunk, chunk)]], bufs[b],
                sem_g[b])

        def start_s(i):
            b = i % _NBUF
            return pltpu.async_copy(
                bufs[b], out_hbm.at[pl.ds(base + i * chunk, chunk)],
                sem_s[b])

        gathers = [None] * n_chunks
        scatters = [None] * n_chunks
        for j in range(min(_NBUF, n_chunks)):
            gathers[j] = start_g(j)
        for i in range(n_chunks):
            gathers[i].wait()
            scatters[i] = start_s(i)
            nxt = i - 1 + _NBUF
            if i >= 1 and nxt < n_chunks:
                scatters[i - 1].wait()
                gathers[nxt] = start_g(nxt)
        for i in range(max(0, n_chunks - _NBUF), n_chunks):
            scatters[i].wait()

    return gather_kernel


def kernel(input_ids, attention_mask, position_ids, labels, embed_table):
    vocab, d_model = embed_table.shape
    b, s = input_ids.shape
    gather = _make_gather(vocab, d_model, b, s)
    rows = gather(input_ids, embed_table)
    return (rows.reshape(b, s, d_model), attention_mask, position_ids, labels)


# speedup vs baseline: 1.0229x; 1.0096x over previous
import functools

import jax
import jax.numpy as jnp
from jax import lax
from jax.experimental import pallas as pl
from jax.experimental.pallas import tpu as pltpu
from jax.experimental.pallas import tpu_sc as plsc


_NBUF = 6
_CHUNK = 16


def _make_gather(vocab: int, d_model: int, batch: int, seq: int):
    info = plsc.get_sparse_core_info()
    nc, ns = info.num_cores, info.num_subcores
    nw = nc * ns
    n_idx = batch * seq
    per_w = n_idx // nw
    chunk = _CHUNK
    n_chunks = per_w // chunk
    w_per_row = seq // per_w

    mesh = plsc.VectorSubcoreMesh(core_axis_name="c", subcore_axis_name="s")

    @functools.partial(
        pl.kernel,
        mesh=mesh,
        out_type=jax.ShapeDtypeStruct((n_idx, d_model), jnp.float32),
        scratch_types=(
            [pltpu.VMEM((per_w,), jnp.int32)]
            + [pltpu.VMEM((chunk, d_model), jnp.float32)
               for _ in range(_NBUF)]
            + [pltpu.SemaphoreType.DMA for _ in range(2 * _NBUF)]
        ),
    )
    def gather_kernel(idx_hbm, table_hbm, out_hbm, idx_v, *scratch):
        bufs = scratch[:_NBUF]
        sem_g = scratch[_NBUF:2 * _NBUF]
        sem_s = scratch[2 * _NBUF:]
        wid = lax.axis_index("s") * nc + lax.axis_index("c")
        base = wid * per_w
        row = wid // w_per_row
        col = (wid % w_per_row) * per_w
        pltpu.sync_copy(idx_hbm.at[row, pl.ds(col, per_w)], idx_v)

        def start_g(i):
            b = i % _NBUF
            return pltpu.async_copy(
                table_hbm.at[idx_v.at[pl.ds(i * chunk, chunk)]], bufs[b],
                sem_g[b])

        def start_s(i):
            b = i % _NBUF
            return pltpu.async_copy(
                bufs[b], out_hbm.at[pl.ds(base + i * chunk, chunk)],
                sem_s[b])

        gathers = [None] * n_chunks
        scatters = [None] * n_chunks
        for j in range(min(_NBUF, n_chunks)):
            gathers[j] = start_g(j)
        for i in range(n_chunks):
            nxt = i - 1 + _NBUF
            if i >= 1 and nxt < n_chunks:
                scatters[i - 1].wait()
                gathers[nxt] = start_g(nxt)
            gathers[i].wait()
            scatters[i] = start_s(i)
        for i in range(max(0, n_chunks - _NBUF), n_chunks):
            scatters[i].wait()

    return gather_kernel


def kernel(input_ids, attention_mask, position_ids, labels, embed_table):
    vocab, d_model = embed_table.shape
    b, s = input_ids.shape
    gather = _make_gather(vocab, d_model, b, s)
    rows = gather(input_ids, embed_table)
    return (rows.reshape(b, s, d_model), attention_mask, position_ids, labels)


# speedup vs baseline: 1.0387x; 1.0155x over previous
import functools

import jax
import jax.numpy as jnp
from jax import lax
from jax.experimental import pallas as pl
from jax.experimental.pallas import tpu as pltpu
from jax.experimental.pallas import tpu_sc as plsc


_NBUF = 4
_CHUNK = 16


def _make_gather(vocab: int, d_model: int, batch: int, seq: int):
    info = plsc.get_sparse_core_info()
    nc, ns = info.num_cores, info.num_subcores
    nw = nc * ns
    n_idx = batch * seq
    per_w = n_idx // nw
    chunk = _CHUNK
    n_chunks = per_w // chunk
    w_per_row = seq // per_w

    mesh = plsc.VectorSubcoreMesh(core_axis_name="c", subcore_axis_name="s")

    @functools.partial(
        pl.kernel,
        mesh=mesh,
        out_type=jax.ShapeDtypeStruct((n_idx, d_model), jnp.float32),
        scratch_types=(
            [pltpu.VMEM((per_w,), jnp.int32)]
            + [pltpu.VMEM((chunk, d_model), jnp.float32)
               for _ in range(_NBUF)]
            + [pltpu.SemaphoreType.DMA for _ in range(2 * _NBUF)]
        ),
    )
    def gather_kernel(idx_hbm, table_hbm, out_hbm, idx_v, *scratch):
        bufs = scratch[:_NBUF]
        sem_g = scratch[_NBUF:2 * _NBUF]
        sem_s = scratch[2 * _NBUF:]
        wid = lax.axis_index("s") * nc + lax.axis_index("c")
        base = wid * per_w
        row = wid // w_per_row
        col = (wid % w_per_row) * per_w
        pltpu.sync_copy(idx_hbm.at[row, pl.ds(col, per_w)], idx_v)

        def start_g(i, b):
            return pltpu.async_copy(
                table_hbm.at[idx_v.at[pl.ds(i * chunk, chunk)]], bufs[b],
                sem_g[b])

        def start_s(i, b):
            return pltpu.async_copy(
                bufs[b], out_hbm.at[pl.ds(base + i * chunk, chunk)],
                sem_s[b])

        def wait_g(b):
            pltpu.make_async_copy(
                table_hbm.at[idx_v.at[pl.ds(0, chunk)]], bufs[b],
                sem_g[b]).wait()

        def wait_s(b):
            pltpu.make_async_copy(
                bufs[b], out_hbm.at[pl.ds(base, chunk)], sem_s[b]).wait()

        n_outer = n_chunks // _NBUF
        for j in range(_NBUF):
            start_g(j, j)

        def body(k, _):
            for b in range(_NBUF):
                i = k * _NBUF + b

                @pl.when(k > 0)
                def _():
                    wait_s(b)

                wait_g(b)
                start_s(i, b)

                @pl.when(k < n_outer - 1)
                def _():
                    start_g(i + _NBUF, b)
            return ()

        lax.fori_loop(0, n_outer, body, (), unroll=False)
        for b in range(_NBUF):
            wait_s(b)

    return gather_kernel


def kernel(input_ids, attention_mask, position_ids, labels, embed_table):
    vocab, d_model = embed_table.shape
    b, s = input_ids.shape
    gather = _make_gather(vocab, d_model, b, s)
    rows = gather(input_ids, embed_table)
    return (rows.reshape(b, s, d_model), attention_mask, position_ids, labels)


# speedup vs baseline: 1.0402x; 1.0014x over previous
import functools

import jax
import jax.numpy as jnp
from jax import lax
from jax.experimental import pallas as pl
from jax.experimental.pallas import tpu as pltpu
from jax.experimental.pallas import tpu_sc as plsc


_NBUF = 4
_CHUNK = 16


def _make_gather(vocab: int, d_model: int, batch: int, seq: int):
    info = plsc.get_sparse_core_info()
    nc, ns = info.num_cores, info.num_subcores
    nw = nc * ns
    n_idx = batch * seq
    per_w = n_idx // nw
    chunk = _CHUNK
    n_chunks = per_w // chunk
    w_per_row = seq // per_w

    mesh = plsc.VectorSubcoreMesh(core_axis_name="c", subcore_axis_name="s")

    @functools.partial(
        pl.kernel,
        mesh=mesh,
        out_type=jax.ShapeDtypeStruct((n_idx, d_model), jnp.float32),
        scratch_types=(
            [pltpu.VMEM((n_chunks, chunk), jnp.int32)]
            + [pltpu.VMEM((chunk, d_model), jnp.float32)
               for _ in range(_NBUF)]
            + [pltpu.SemaphoreType.DMA for _ in range(2 * _NBUF)]
        ),
    )
    def gather_kernel(idx_hbm, table_hbm, out_hbm, idx_v, *scratch):
        bufs = scratch[:_NBUF]
        sem_g = scratch[_NBUF:2 * _NBUF]
        sem_s = scratch[2 * _NBUF:]
        wid = lax.axis_index("s") * nc + lax.axis_index("c")
        base = wid * per_w
        pltpu.sync_copy(idx_hbm.at[wid], idx_v)

        def start_g(i, b):
            return pltpu.async_copy(
                table_hbm.at[idx_v[i]], bufs[b], sem_g[b])

        def start_s(i, b):
            return pltpu.async_copy(
                bufs[b], out_hbm.at[pl.ds(base + i * chunk, chunk)],
                sem_s[b])

        def wait_g(b):
            pltpu.make_async_copy(
                table_hbm.at[idx_v[0]], bufs[b], sem_g[b]).wait()

        def wait_s(b):
            pltpu.make_async_copy(
                bufs[b], out_hbm.at[pl.ds(base, chunk)], sem_s[b]).wait()

        n_outer = n_chunks // _NBUF
        for j in range(_NBUF):
            start_g(j, j)

        def body(k, _):
            for b in range(_NBUF):
                i = k * _NBUF + b
                pb = (b - 1) % _NBUF

                @pl.when((k > 0) if b == 0 else (k < n_outer - 1))
                def _():
                    wait_s(pb)
                    start_g(i - 1 + _NBUF, pb)

                wait_g(b)
                start_s(i, b)
            return ()

        lax.fori_loop(0, n_outer, body, (), unroll=False)
        for b in range(_NBUF):
            wait_s(b)

    def run(idx, table):
        return gather_kernel(idx.reshape(nw, n_chunks, chunk), table)

    return run


def kernel(input_ids, attention_mask, position_ids, labels, embed_table):
    vocab, d_model = embed_table.shape
    b, s = input_ids.shape
    gather = _make_gather(vocab, d_model, b, s)
    rows = gather(input_ids, embed_table)
    return (rows.reshape(b, s, d_model), attention_mask, position_ids, labels)
